# baseline (speedup 1.0000x reference)
import functools

import jax
import jax.numpy as jnp
from jax import lax
from jax.experimental import pallas as pl
from jax.experimental.pallas import tpu as pltpu
from jax.experimental.pallas import tpu_sc as plsc

_K = 256
_B = 128
_N = 32768
_NC = 2
_NS = 16
_NW = _NC * _NS
_ROWS_PER_W = _B // _NW
_NCHUNK = _N // 16
_CCAP = 16384
_SCAP = 512
_ZCHUNK = 8192


def _lane():
    return lax.iota(jnp.int32, 16)


def _splat0(v):
    return jnp.sum(jnp.where(_lane() == 0, v, 0))


def _mono_i32(bits):
    return jnp.where(bits >= 0, bits, bits ^ jnp.int32(0x7FFFFFFF))


def _clear_hist(hist_v):
    def body(c, _):
        hist_v[pl.ds(c * 16, 16)] = jnp.zeros((16,), jnp.int32)
        return 0

    lax.fori_loop(0, 256, body, 0, unroll=4)


def _scan_hist(hist_v, target):
    def body(t, st):
        carry, b, above = st
        d = 255 - t
        tot = jnp.sum(hist_v[pl.ds(d * 16, 16)])
        new_carry = carry + tot
        found = jnp.logical_and(new_carry >= target, b < 0)
        b = jnp.where(found, d, b)
        above = jnp.where(found, carry, above)
        return (new_carry, b, above)

    _, b, above = lax.fori_loop(
        0, 256, body, (jnp.int32(0), jnp.int32(-1), jnp.int32(0)), unroll=4
    )
    return b, above


def _sc_body(scores_hbm, out_hbm, xv, mv, hist_v, cval, cidx,
             sem_in, sem_sc):
    wid = lax.axis_index("s") * _NC + lax.axis_index("c")
    lane = _lane()
    ones16 = jnp.ones((16,), jnp.int32)

    def row_body(j, _):
        row = wid * _ROWS_PER_W + j
        rbase = row * _N

        in_copy = pltpu.make_async_copy(
            scores_hbm.at[pl.ds(rbase, _N)], xv, sem_in
        )
        in_copy.start()
        in_copy.wait()

        _clear_hist(hist_v)

        def l0_body(i, _):
            skey = _mono_i32(xv[pl.ds(i * 16, 16)])
            digit = (skey >> 24) + 128
            plsc.addupdate_scatter(hist_v, [digit * 16 + lane], ones16)
            return 0

        lax.fori_loop(0, _NCHUNK, l0_body, 0, unroll=8)

        b0, above0 = _scan_hist(hist_v, jnp.int32(_K))
        kk = _K - above0
        thr0 = (b0 - 128) << 24

        def comp_body(i, off):
            skey = _mono_i32(xv[pl.ds(i * 16, 16)])
            keep = skey >= thr0
            pos = off + plsc.cumsum(keep.astype(jnp.int32)) - 1
            keep = jnp.logical_and(keep, pos < _CCAP)
            plsc.store_scatter(cval, [pos], skey, mask=keep)
            plsc.store_scatter(cidx, [pos], i * 16 + lane, mask=keep)
            return off + plsc.all_reduce_population_count(keep)

        moff = lax.fori_loop(
            0, _NCHUNK, comp_body, jnp.zeros((16,), jnp.int32), unroll=8
        )
        m = jnp.max(moff)
        nci = (m + 15) >> 4

        prefix = b0 - 128
        for lvl in (1, 2, 3):
            shift = 24 - 8 * lvl
            _clear_hist(hist_v)

            def rf_body(i, _, shift=shift, prefix=prefix):
                skey = cval[pl.ds(i * 16, 16)]
                valid = (i * 16 + lane) < m
                keep = jnp.logical_and((skey >> (shift + 8)) == prefix, valid)
                digit = (skey >> shift) & 255
                plsc.addupdate_scatter(
                    hist_v, [digit * 16 + lane], ones16, mask=keep
                )
                return 0

            lax.fori_loop(0, nci, rf_body, 0)
            bl, abovel = _scan_hist(hist_v, kk)
            kk = kk - abovel
            prefix = (prefix << 8) | bl

        thr = prefix

        one16f = jnp.ones((16,), jnp.float32)
        zero16f = jnp.zeros((16,), jnp.float32)

        def mask_body(i, _):
            skey = _mono_i32(xv[pl.ds(i * 16, 16)])
            mv[pl.ds(i * 16, 16)] = jnp.where(skey >= thr, one16f, zero16f)
            return 0

        lax.fori_loop(0, _NCHUNK, mask_body, 0, unroll=8)

        out_copy = pltpu.make_async_copy(mv, out_hbm.at[pl.ds(rbase, _N)], sem_sc)
        out_copy.start()
        out_copy.wait()
        return 0

    lax.fori_loop(0, _ROWS_PER_W, row_body, 0)


@functools.partial(jax.jit, donate_argnums=())
def kernel(scores):
    b, n = scores.shape
    flat = lax.bitcast_convert_type(scores, jnp.int32).reshape(b * n)
    mesh = plsc.VectorSubcoreMesh(core_axis_name="c", subcore_axis_name="s")
    run = pl.kernel(
        _sc_body,
        out_type=jax.ShapeDtypeStruct((b * n,), jnp.float32),
        mesh=mesh,
        scratch_types=[
            pltpu.VMEM((_N,), jnp.int32),
            pltpu.VMEM((_N,), jnp.float32),
            pltpu.VMEM((4096,), jnp.int32),
            pltpu.VMEM((_CCAP,), jnp.int32),
            pltpu.VMEM((_CCAP,), jnp.int32),
            pltpu.SemaphoreType.DMA,
            pltpu.SemaphoreType.DMA,
        ],
        compiler_params=pltpu.CompilerParams(needs_layout_passes=False),
    )
    return run(flat).reshape(b, n)
